# Initial kernel scaffold; baseline (speedup 1.0000x reference)
import functools

import jax
import jax.numpy as jnp
from jax import lax
from jax.experimental import pallas as pl
from jax.experimental.pallas import tpu as pltpu
from jax.experimental.pallas import tpu_sc as plsc

N = 10000
NP = 10240
C = 128
E = 320000
NSC = 2
NSUB = 16
NW = NSC * NSUB
CH = 128
NCHT = 79
EPT = NCHT * CH
EP = NW * EPT
RB = 512
NEG_SLOPE_CONST = 0.2


def _prolog_body(x_ref, w_ref, att_ref, xp_ref, al_ref):
    xb = x_ref[...]
    xp = jnp.dot(xb, w_ref[...], preferred_element_type=jnp.float32)
    xp_ref[...] = xp
    al_ref[...] = lax.dot_general(
        att_ref[...], xp, (((1,), (1,)), ((), ())),
        preferred_element_type=jnp.float32)


def _prolog(x_pad, W, att8):
    return pl.pallas_call(
        _prolog_body,
        grid=(NP // RB,),
        in_specs=[
            pl.BlockSpec((RB, C), lambda i: (i, 0)),
            pl.BlockSpec((C, C), lambda i: (0, 0)),
            pl.BlockSpec((8, C), lambda i: (0, 0)),
        ],
        out_specs=[
            pl.BlockSpec((RB, C), lambda i: (i, 0)),
            pl.BlockSpec((8, RB), lambda i: (0, i)),
        ],
        out_shape=[
            jax.ShapeDtypeStruct((NP, C), jnp.float32),
            jax.ShapeDtypeStruct((8, NP), jnp.float32),
        ],
    )(x_pad, W, att8)


def _sc_body(as_hbm, ad_hbm, xp_hbm, src_hbm, dst_hbm,
             outp_hbm, den_hbm,
             as_v, ad_v, src_v, dst_v, ex_v, den_v, rows_v, out_sh):
    cid = lax.axis_index("c")
    sid = lax.axis_index("s")
    wid = cid * NSUB + sid

    pltpu.sync_copy(as_hbm, as_v)
    pltpu.sync_copy(ad_hbm, ad_v)
    pltpu.sync_copy(src_hbm.at[pl.ds(wid * EPT, EPT)], src_v)
    pltpu.sync_copy(dst_hbm.at[wid], dst_v)

    zero16 = jnp.zeros((16,), jnp.float32)

    @pl.loop(0, NP, step=16)
    def _(i):
        den_v[pl.ds(i, 16)] = zero16

    @pl.loop(0, CH)
    def _(r):
        @pl.loop(0, C, step=16)
        def _(c0):
            rows_v[r, pl.ds(c0, 16)] = zero16

    @pl.loop(0, NP // NSUB // CH)
    def _(b):
        pltpu.sync_copy(rows_v, out_sh.at[pl.ds(sid * (NP // NSUB) + b * CH, CH)])
    plsc.subcore_barrier()

    @pl.loop(0, NCHT)
    def _(j):
        @pl.loop(0, CH, step=16)
        def _(k):
            s_idx = src_v[pl.ds(j * CH + k, 16)]
            d_idx = dst_v[j, pl.ds(k, 16)]
            a = plsc.load_gather(as_v, [s_idx]) + plsc.load_gather(ad_v, [d_idx])
            a = jnp.maximum(a, a * NEG_SLOPE_CONST)
            e = jnp.exp(a)
            ex_v[pl.ds(j * CH + k, 16)] = e
            plsc.addupdate_scatter(den_v, [d_idx], e)

    pltpu.sync_copy(den_v, den_hbm.at[wid])

    @pl.loop(0, NCHT)
    def _(j):
        pltpu.sync_copy(xp_hbm.at[src_v.at[pl.ds(j * CH, CH)]], rows_v)

        @pl.loop(0, CH)
        def _(r):
            ridx = jnp.full((16,), 0, jnp.int32) + (j * CH + r)
            ev = plsc.load_gather(ex_v, [ridx])

            @pl.loop(0, C, step=16)
            def _(c0):
                rows_v[r, pl.ds(c0, 16)] = rows_v[r, pl.ds(c0, 16)] * ev

        pltpu.sync_copy(rows_v, out_sh.at[dst_v.at[j]], add=True)

    plsc.subcore_barrier()

    @pl.loop(0, NP // NSUB // CH)
    def _(b):
        base = sid * (NP // NSUB) + b * CH
        pltpu.sync_copy(out_sh.at[pl.ds(base, CH)],
                        outp_hbm.at[cid, pl.ds(base, CH)])


def _sc_stage(as_arr, ad_arr, xp, src_p, dst3):
    mesh = plsc.VectorSubcoreMesh(core_axis_name="c", subcore_axis_name="s")
    k = pl.kernel(
        _sc_body,
        out_type=[
            jax.ShapeDtypeStruct((NSC, NP, C), jnp.float32),
            jax.ShapeDtypeStruct((NW, NP), jnp.float32),
        ],
        mesh=mesh,
        scratch_types=[
            pltpu.VMEM((NP,), jnp.float32),
            pltpu.VMEM((NP,), jnp.float32),
            pltpu.VMEM((EPT,), jnp.int32),
            pltpu.VMEM((NCHT, CH), jnp.int32),
            pltpu.VMEM((EPT,), jnp.float32),
            pltpu.VMEM((NP,), jnp.float32),
            pltpu.VMEM((CH, C), jnp.float32),
            pltpu.VMEM_SHARED((NP, C), jnp.float32),
        ],
    )
    return k(as_arr, ad_arr, xp, src_p, dst3)


def _epilog_body(p_ref, d_ref, b_ref, o_ref):
    s = p_ref[0, :, :] + p_ref[1, :, :]
    ones = jnp.ones((NW, 1), jnp.float32)
    dn = lax.dot_general(d_ref[...], ones, (((0,), (0,)), ((), ())),
                         preferred_element_type=jnp.float32)
    o_ref[...] = s / (dn + 1e-16) + b_ref[...]


def _epilog(parts, denp, bias2):
    return pl.pallas_call(
        _epilog_body,
        grid=(NP // RB,),
        in_specs=[
            pl.BlockSpec((NSC, RB, C), lambda i: (0, i, 0)),
            pl.BlockSpec((NW, RB), lambda i: (0, i)),
            pl.BlockSpec((1, C), lambda i: (0, 0)),
        ],
        out_specs=pl.BlockSpec((RB, C), lambda i: (i, 0)),
        out_shape=jax.ShapeDtypeStruct((NP, C), jnp.float32),
    )(parts, denp, bias2)


def kernel(x, edge_index, W, att_src, att_dst, bias):
    x_pad = jnp.pad(x, ((0, NP - N), (0, 0)))
    att8 = jnp.concatenate(
        [att_src.reshape(1, C), att_dst.reshape(1, C),
         jnp.zeros((6, C), jnp.float32)], axis=0)
    src = edge_index[0]
    dst = edge_index[1]
    pad_idx = jnp.full((EP - E,), N, jnp.int32)
    src_p = jnp.concatenate([src, pad_idx])
    dst3 = jnp.concatenate([dst, pad_idx]).reshape(NW, NCHT, CH)
    bias2 = bias.reshape(1, C)

    xp, alpha = _prolog(x_pad, W, att8)
    as_arr = alpha[0]
    ad_arr = alpha[1]
    parts, denp = _sc_stage(as_arr, ad_arr, xp, src_p, dst3)
    out = _epilog(parts, denp, bias2)
    return out[:N]


# speedup vs baseline: 18.9601x; 18.9601x over previous
import dataclasses
import functools

import jax
import jax.numpy as jnp
from jax import lax
from jax.experimental import pallas as pl
from jax.experimental.pallas import tpu as pltpu
from jax.experimental.pallas import tpu_sc as plsc

N = 10000
NP = 10240
C = 128
E = 320000
NSC = 2
NSUB = 16
NW = NSC * NSUB
CH = 128
NCHT = 79
EPT = NCHT * CH
EP = NW * EPT
RB = 512
NEG_SLOPE_CONST = 0.2


def _prolog_body(x_ref, w_ref, att_ref, xp_ref, al_ref):
    xb = x_ref[...]
    xp = jnp.dot(xb, w_ref[...], preferred_element_type=jnp.float32)
    xp_ref[...] = xp
    al_ref[...] = lax.dot_general(
        att_ref[...], xp, (((1,), (1,)), ((), ())),
        preferred_element_type=jnp.float32)


def _prolog(x_pad, W, att8):
    return pl.pallas_call(
        _prolog_body,
        grid=(NP // RB,),
        in_specs=[
            pl.BlockSpec((RB, C), lambda i: (i, 0)),
            pl.BlockSpec((C, C), lambda i: (0, 0)),
            pl.BlockSpec((8, C), lambda i: (0, 0)),
        ],
        out_specs=[
            pl.BlockSpec((RB, C), lambda i: (i, 0)),
            pl.BlockSpec((8, RB), lambda i: (0, i)),
        ],
        out_shape=[
            jax.ShapeDtypeStruct((NP, C), jnp.float32),
            jax.ShapeDtypeStruct((8, NP), jnp.float32),
        ],
    )(x_pad, W, att8)


NR = 10112
RPT = NR // NSUB


def _sc_a_body(as_hbm, ad_hbm, src_hbm, dst_hbm,
               den_hbm, ex_hbm,
               as_v, ad_v, src_v, dst_v, ex_v, den_v):
    cid = lax.axis_index("c")
    sid = lax.axis_index("s")
    wid = cid * NSUB + sid

    pltpu.sync_copy(as_hbm, as_v)
    pltpu.sync_copy(ad_hbm, ad_v)
    pltpu.sync_copy(src_hbm.at[pl.ds(wid * EPT, EPT)], src_v)
    pltpu.sync_copy(dst_hbm.at[pl.ds(wid * EPT, EPT)], dst_v)

    zero16 = jnp.zeros((16,), jnp.float32)

    @pl.loop(0, NP, step=16)
    def _(i):
        den_v[pl.ds(i, 16)] = zero16

    @pl.loop(0, EPT, step=16)
    def _(i):
        s_idx = src_v[pl.ds(i, 16)]
        d_idx = dst_v[pl.ds(i, 16)]
        a = plsc.load_gather(as_v, [s_idx]) + plsc.load_gather(ad_v, [d_idx])
        a = jnp.maximum(a, a * NEG_SLOPE_CONST)
        e = jnp.exp(a)
        ex_v[pl.ds(i, 16)] = e
        plsc.addupdate_scatter(den_v, [d_idx], e)

    pltpu.sync_copy(den_v, den_hbm.at[wid])
    pltpu.sync_copy(ex_v, ex_hbm.at[pl.ds(wid * EPT, EPT)])


def _sc_b_body(xp_hbm, src_hbm, dst_hbm, ex_hbm, zeros_hbm,
               outp_hbm,
               src_v, dst_v, ex_v, rows_v, out_sh):
    cid = lax.axis_index("c")
    sid = lax.axis_index("s")
    wid = cid * NSUB + sid

    pltpu.sync_copy(src_hbm.at[pl.ds(wid * EPT, EPT)], src_v)
    pltpu.sync_copy(dst_hbm.at[wid], dst_v)
    pltpu.sync_copy(ex_hbm.at[pl.ds(wid * EPT, EPT)], ex_v)

    pltpu.sync_copy(zeros_hbm, out_sh.at[pl.ds(sid * RPT, RPT)])
    plsc.subcore_barrier()

    @pl.loop(0, NCHT)
    def _(j):
        pltpu.sync_copy(xp_hbm.at[src_v.at[pl.ds(j * CH, CH)]], rows_v)

        @pl.loop(0, CH)
        def _(r):
            ridx = jnp.full((16,), 0, jnp.int32) + (j * CH + r)
            ev = plsc.load_gather(ex_v, [ridx])

            @pl.loop(0, C, step=16)
            def _(c0):
                rows_v[r, pl.ds(c0, 16)] = rows_v[r, pl.ds(c0, 16)] * ev

        pltpu.sync_copy(rows_v, out_sh.at[dst_v.at[j]], add=True)

    plsc.subcore_barrier()
    pltpu.sync_copy(out_sh.at[pl.ds(sid * RPT, RPT)],
                    outp_hbm.at[cid, pl.ds(sid * RPT, RPT)])


def _sc_compiler_params():
    cp = pltpu.CompilerParams()
    if "needs_layout_passes" in pltpu.CompilerParams.__dataclass_fields__:
        cp = dataclasses.replace(cp, needs_layout_passes=False)
    return cp


def _sc_stage(as_arr, ad_arr, xp, src_p, dst_p, dst3, zeros_b):
    mesh = plsc.VectorSubcoreMesh(core_axis_name="c", subcore_axis_name="s")
    cp = _sc_compiler_params()
    ka = pl.kernel(
        _sc_a_body,
        out_type=[
            jax.ShapeDtypeStruct((NW, NP), jnp.float32),
            jax.ShapeDtypeStruct((EP,), jnp.float32),
        ],
        mesh=mesh,
        compiler_params=cp,
        scratch_types=[
            pltpu.VMEM((NP,), jnp.float32),
            pltpu.VMEM((NP,), jnp.float32),
            pltpu.VMEM((EPT,), jnp.int32),
            pltpu.VMEM((EPT,), jnp.int32),
            pltpu.VMEM((EPT,), jnp.float32),
            pltpu.VMEM((NP,), jnp.float32),
        ],
    )
    denp, ex = ka(as_arr, ad_arr, src_p, dst_p)
    kb = pl.kernel(
        _sc_b_body,
        out_type=jax.ShapeDtypeStruct((NSC, NP, C), jnp.float32),
        mesh=mesh,
        compiler_params=cp,
        scratch_types=[
            pltpu.VMEM((EPT,), jnp.int32),
            pltpu.VMEM((NCHT, CH), jnp.int32),
            pltpu.VMEM((EPT,), jnp.float32),
            pltpu.VMEM((CH, C), jnp.float32),
            pltpu.VMEM_SHARED((NR, C), jnp.float32),
        ],
    )
    parts = kb(xp, src_p, dst3, ex, zeros_b)
    return parts, denp


def _epilog_body(p_ref, d_ref, b_ref, o_ref):
    s = p_ref[0, :, :] + p_ref[1, :, :]
    ones = jnp.ones((NW, 1), jnp.float32)
    dn = lax.dot_general(d_ref[...], ones, (((0,), (0,)), ((), ())),
                         preferred_element_type=jnp.float32)
    o_ref[...] = s / (dn + 1e-16) + b_ref[...]


def _epilog(parts, denp, bias2):
    return pl.pallas_call(
        _epilog_body,
        grid=(NP // RB,),
        in_specs=[
            pl.BlockSpec((NSC, RB, C), lambda i: (0, i, 0)),
            pl.BlockSpec((NW, RB), lambda i: (0, i)),
            pl.BlockSpec((1, C), lambda i: (0, 0)),
        ],
        out_specs=pl.BlockSpec((RB, C), lambda i: (i, 0)),
        out_shape=jax.ShapeDtypeStruct((NP, C), jnp.float32),
    )(parts, denp, bias2)


def kernel(x, edge_index, W, att_src, att_dst, bias):
    x_pad = jnp.pad(x, ((0, NP - N), (0, 0)))
    att8 = jnp.concatenate(
        [att_src.reshape(1, C), att_dst.reshape(1, C),
         jnp.zeros((6, C), jnp.float32)], axis=0)
    src = edge_index[0]
    dst = edge_index[1]
    pad_idx = jnp.full((EP - E,), N, jnp.int32)
    src_p = jnp.concatenate([src, pad_idx])
    dst_p = jnp.concatenate([dst, pad_idx])
    dst3 = dst_p.reshape(NW, NCHT, CH)
    zeros_b = jnp.zeros((RPT, C), jnp.float32)
    bias2 = bias.reshape(1, C)

    xp, alpha = _prolog(x_pad, W, att8)
    as_arr = alpha[0]
    ad_arr = alpha[1]
    parts, denp = _sc_stage(as_arr, ad_arr, xp, src_p, dst_p, dst3, zeros_b)
    out = _epilog(parts, denp, bias2)
    return out[:N]
